# Initial kernel scaffold; baseline (speedup 1.0000x reference)
import jax
import jax.numpy as jnp
from jax.experimental import pallas as pl


def kernel(x, W_enc, b_enc, W_dec, b_dec):
    raise NotImplementedError("write your pallas kernel here")


# speedup vs baseline: 1.1298x; 1.1298x over previous
import functools

import jax
import jax.numpy as jnp
from jax.experimental import pallas as pl
from jax.experimental.pallas import tpu as pltpu

_B = 2048
_A = 2048
_D = 16384
_K = 64

_BN_ENC = 512
_BD_DEC = 256


def _enc_body(x_ref, w_ref, b_ref, z_ref):
    acc = jax.lax.dot_general(
        x_ref[...], w_ref[...], (((1,), (1,)), ((), ())),
        preferred_element_type=jnp.float32)
    z_ref[...] = acc + b_ref[...]


def _encode(x, W_enc, b_enc):
    return pl.pallas_call(
        _enc_body,
        grid=(_D // _BN_ENC,),
        in_specs=[
            pl.BlockSpec((_B, _A), lambda j: (0, 0)),
            pl.BlockSpec((_BN_ENC, _A), lambda j: (j, 0)),
            pl.BlockSpec((1, _BN_ENC), lambda j: (0, j)),
        ],
        out_specs=pl.BlockSpec((_B, _BN_ENC), lambda j: (0, j)),
        out_shape=jax.ShapeDtypeStruct((_B, _D), jnp.float32),
    )(x, W_enc, b_enc.reshape(1, _D))


def _dec_body(t_ref, z_ref, w_ref, b_ref, xhat_ref, sz_ref, mask_ref):
    j = pl.program_id(0)
    z = z_ref[...]
    bits = jax.lax.bitcast_convert_type(z, jnp.int32) & jnp.int32(0x7FFFFFFF)
    m = bits >= t_ref[0]
    sz = jnp.where(m, z, 0.0)
    sz_ref[...] = sz
    mask_ref[...] = m
    acc = jax.lax.dot_general(
        sz, w_ref[...], (((1,), (1,)), ((), ())),
        preferred_element_type=jnp.float32)

    @pl.when(j == 0)
    def _():
        xhat_ref[...] = acc + b_ref[...]

    @pl.when(j > 0)
    def _():
        xhat_ref[...] += acc


def _decode(z, W_dec, b_dec, thresh_bits):
    return pl.pallas_call(
        _dec_body,
        grid=(_D // _BD_DEC,),
        in_specs=[
            pl.BlockSpec(memory_space=pltpu.MemorySpace.SMEM),
            pl.BlockSpec((_B, _BD_DEC), lambda j: (0, j)),
            pl.BlockSpec((_A, _BD_DEC), lambda j: (0, j)),
            pl.BlockSpec((1, _A), lambda j: (0, 0)),
        ],
        out_specs=[
            pl.BlockSpec((_B, _A), lambda j: (0, 0)),
            pl.BlockSpec((_B, _BD_DEC), lambda j: (0, j)),
            pl.BlockSpec((_B, _BD_DEC), lambda j: (0, j)),
        ],
        out_shape=[
            jax.ShapeDtypeStruct((_B, _A), jnp.float32),
            jax.ShapeDtypeStruct((_B, _D), jnp.float32),
            jax.ShapeDtypeStruct((_B, _D), jnp.bool_),
        ],
    )(thresh_bits, z, W_dec, b_dec.reshape(1, _A))


def kernel(x, W_enc, b_enc, W_dec, b_dec):
    z = _encode(x, W_enc, b_enc)
    abs_bits = jax.lax.bitcast_convert_type(z, jnp.int32) & jnp.int32(0x7FFFFFFF)
    k_total = _K * _B
    topv, _ = jax.lax.top_k(abs_bits.reshape(-1), k_total)
    thresh = topv[k_total - 1].reshape(1)
    x_hat, sparse_z, mask = _decode(z, W_dec, b_dec, thresh)
    return x_hat, sparse_z, mask


# speedup vs baseline: 26.7029x; 23.6348x over previous
import functools

import jax
import jax.numpy as jnp
from jax import lax
from jax.experimental import pallas as pl
from jax.experimental.pallas import tpu as pltpu
from jax.experimental.pallas import tpu_sc as plsc

_B = 2048
_A = 2048
_D = 16384
_K = 64
_KT = _K * _B
_N = _B * _D

_BN_ENC = 512
_BD_DEC = 256

_NW = 32
_PER_W = _N // _NW
_CHUNK = 16384
_H1 = 65536
_H2 = 32768

_mesh = plsc.VectorSubcoreMesh(core_axis_name="c", subcore_axis_name="s")


def _wid():
    return lax.axis_index("s") * 2 + lax.axis_index("c")


def _zero_ref(ref, n):
    zeros = jnp.zeros((16,), jnp.int32)

    def body(i, _):
        ref[pl.ds(i * 16, 16)] = zeros
        return 0

    lax.fori_loop(0, n // 16, body, 0)


@functools.partial(
    pl.kernel,
    out_type=jax.ShapeDtypeStruct((_NW, _H1), jnp.int32),
    mesh=_mesh,
    compiler_params=pltpu.CompilerParams(needs_layout_passes=False),
    scratch_types=[
        pltpu.VMEM((_CHUNK,), jnp.int32),
        pltpu.VMEM((_H1,), jnp.int32),
    ],
)
def _sc_hist1(z_hbm, out_hbm, buf, hist):
    wid = _wid()
    base = wid * _PER_W
    _zero_ref(hist, _H1)
    ones = jnp.ones((16,), jnp.int32)
    mask31 = jnp.int32(0x7FFFFFFF)

    def chunk_body(c, _):
        pltpu.sync_copy(z_hbm.at[pl.ds(base + c * _CHUNK, _CHUNK)], buf)

        def vec_body(i, _):
            b = buf[pl.ds(i * 16, 16)] & mask31
            idx = lax.shift_right_logical(b, 15)
            plsc.addupdate_scatter(hist, [idx], ones)
            return 0

        lax.fori_loop(0, _CHUNK // 16, vec_body, 0, unroll=4)
        return 0

    lax.fori_loop(0, _PER_W // _CHUNK, chunk_body, 0)
    pltpu.sync_copy(hist, out_hbm.at[wid])


@functools.partial(
    pl.kernel,
    out_type=jax.ShapeDtypeStruct((_NW, _H2), jnp.int32),
    mesh=_mesh,
    compiler_params=pltpu.CompilerParams(needs_layout_passes=False),
    scratch_types=[
        pltpu.VMEM((_CHUNK,), jnp.int32),
        pltpu.VMEM((_H2,), jnp.int32),
        pltpu.VMEM((16,), jnp.int32),
    ],
)
def _sc_hist2(z_hbm, b1_hbm, out_hbm, buf, hist, b1buf):
    wid = _wid()
    base = wid * _PER_W
    _zero_ref(hist, _H2)
    pltpu.sync_copy(b1_hbm, b1buf)
    b1 = b1buf[...]
    ones = jnp.ones((16,), jnp.int32)
    mask31 = jnp.int32(0x7FFFFFFF)
    mask15 = jnp.int32(0x7FFF)

    def chunk_body(c, _):
        pltpu.sync_copy(z_hbm.at[pl.ds(base + c * _CHUNK, _CHUNK)], buf)

        def vec_body(i, _):
            b = buf[pl.ds(i * 16, 16)] & mask31
            m = lax.shift_right_logical(b, 15) == b1
            idx = b & mask15
            plsc.addupdate_scatter(hist, [idx], ones, mask=m)
            return 0

        lax.fori_loop(0, _CHUNK // 16, vec_body, 0, unroll=4)
        return 0

    lax.fori_loop(0, _PER_W // _CHUNK, chunk_body, 0)
    pltpu.sync_copy(hist, out_hbm.at[wid])


def _pick1_body(h_ref, b1_ref, r_ref):
    hs = jnp.sum(h_ref[...], axis=0, keepdims=True)
    iota = lax.broadcasted_iota(jnp.int32, (1, _H1), 1)
    kt = jnp.int32(_KT)

    def cnt(m):
        return jnp.sum(jnp.where(iota >= m, hs, 0))

    def body(t, lohi):
        lo, hi = lohi
        mid = (lo + hi) // 2
        p = cnt(mid) >= kt
        return (jnp.where(p, mid, lo), jnp.where(p, hi, mid))

    lo, _ = lax.fori_loop(0, 16, body, (jnp.int32(0), jnp.int32(_H1)))
    r = kt - cnt(lo + 1)
    b1_ref[...] = jnp.full((8, 128), lo, jnp.int32)
    r_ref[...] = jnp.full((8, 128), r, jnp.int32)


def _pick1(hist1):
    return pl.pallas_call(
        _pick1_body,
        in_specs=[pl.BlockSpec((_NW, _H1), lambda: (0, 0))],
        out_specs=[pl.BlockSpec((8, 128), lambda: (0, 0))] * 2,
        out_shape=[jax.ShapeDtypeStruct((8, 128), jnp.int32)] * 2,
    )(hist1)


def _pick2_body(h_ref, b1_ref, r_ref, t_ref):
    hs = jnp.sum(h_ref[...], axis=0, keepdims=True)
    iota = lax.broadcasted_iota(jnp.int32, (1, _H2), 1)
    r = r_ref[0, 0]
    b1 = b1_ref[0, 0]

    def cnt(m):
        return jnp.sum(jnp.where(iota >= m, hs, 0))

    def body(t, lohi):
        lo, hi = lohi
        mid = (lo + hi) // 2
        p = cnt(mid) >= r
        return (jnp.where(p, mid, lo), jnp.where(p, hi, mid))

    lo, _ = lax.fori_loop(0, 15, body, (jnp.int32(0), jnp.int32(_H2)))
    t = lax.shift_left(b1, 15) | lo
    t_ref[...] = jnp.full((8, 128), t, jnp.int32)


def _pick2(hist2, b1s, rs):
    return pl.pallas_call(
        _pick2_body,
        in_specs=[
            pl.BlockSpec((_NW, _H2), lambda: (0, 0)),
            pl.BlockSpec(memory_space=pltpu.MemorySpace.SMEM),
            pl.BlockSpec(memory_space=pltpu.MemorySpace.SMEM),
        ],
        out_specs=pl.BlockSpec((8, 128), lambda: (0, 0)),
        out_shape=jax.ShapeDtypeStruct((8, 128), jnp.int32),
    )(hist2, b1s, rs)


def _enc_body(x_ref, w_ref, b_ref, z_ref):
    acc = lax.dot_general(
        x_ref[...], w_ref[...], (((1,), (1,)), ((), ())),
        preferred_element_type=jnp.float32)
    z_ref[...] = acc + b_ref[...]


def _encode(x, W_enc, b_enc):
    return pl.pallas_call(
        _enc_body,
        grid=(_D // _BN_ENC,),
        in_specs=[
            pl.BlockSpec((_B, _A), lambda j: (0, 0)),
            pl.BlockSpec((_BN_ENC, _A), lambda j: (j, 0)),
            pl.BlockSpec((1, _BN_ENC), lambda j: (0, j)),
        ],
        out_specs=pl.BlockSpec((_B, _BN_ENC), lambda j: (0, j)),
        out_shape=jax.ShapeDtypeStruct((_B, _D), jnp.float32),
    )(x, W_enc, b_enc.reshape(1, _D))


def _dec_body(t_ref, z_ref, w_ref, b_ref, xhat_ref, sz_ref, mask_ref):
    j = pl.program_id(0)
    z = z_ref[...]
    bits = lax.bitcast_convert_type(z, jnp.int32) & jnp.int32(0x7FFFFFFF)
    m = bits >= t_ref[0]
    sz = jnp.where(m, z, 0.0)
    sz_ref[...] = sz
    mask_ref[...] = m
    acc = lax.dot_general(
        sz, w_ref[...], (((1,), (1,)), ((), ())),
        preferred_element_type=jnp.float32)

    @pl.when(j == 0)
    def _():
        xhat_ref[...] = acc + b_ref[...]

    @pl.when(j > 0)
    def _():
        xhat_ref[...] += acc


def _decode(z, W_dec, b_dec, thresh_bits):
    return pl.pallas_call(
        _dec_body,
        grid=(_D // _BD_DEC,),
        in_specs=[
            pl.BlockSpec(memory_space=pltpu.MemorySpace.SMEM),
            pl.BlockSpec((_B, _BD_DEC), lambda j: (0, j)),
            pl.BlockSpec((_A, _BD_DEC), lambda j: (0, j)),
            pl.BlockSpec((1, _A), lambda j: (0, 0)),
        ],
        out_specs=[
            pl.BlockSpec((_B, _A), lambda j: (0, 0)),
            pl.BlockSpec((_B, _BD_DEC), lambda j: (0, j)),
            pl.BlockSpec((_B, _BD_DEC), lambda j: (0, j)),
        ],
        out_shape=[
            jax.ShapeDtypeStruct((_B, _A), jnp.float32),
            jax.ShapeDtypeStruct((_B, _D), jnp.float32),
            jax.ShapeDtypeStruct((_B, _D), jnp.bool_),
        ],
    )(thresh_bits, z, W_dec, b_dec.reshape(1, _A))


def kernel(x, W_enc, b_enc, W_dec, b_dec):
    z = _encode(x, W_enc, b_enc)
    zf = lax.bitcast_convert_type(z, jnp.int32).reshape(-1)
    hist1 = _sc_hist1(zf)
    b1_full, r_full = _pick1(hist1)
    b1vec = b1_full[0, :16]
    hist2 = _sc_hist2(zf, b1vec)
    t_full = _pick2(hist2, b1_full[:1, :1], r_full[:1, :1])
    x_hat, sparse_z, mask = _decode(z, W_dec, b_dec, t_full[0, :1])
    return x_hat, sparse_z, mask


# speedup vs baseline: 31.0274x; 1.1619x over previous
import functools

import jax
import jax.numpy as jnp
from jax import lax
from jax.experimental import pallas as pl
from jax.experimental.pallas import tpu as pltpu
from jax.experimental.pallas import tpu_sc as plsc

_B = 2048
_A = 2048
_D = 16384
_K = 64
_KT = _K * _B
_N = _B * _D

_BN_ENC = 512
_BD_DEC = 256

_NW = 32
_PER_W = _N // _NW
_CHUNK = 16384
_NPAIR = _PER_W // (2 * _CHUNK)
_H1 = 65536
_H2 = 32768

_mesh = plsc.VectorSubcoreMesh(core_axis_name="c", subcore_axis_name="s")
_sc_params = pltpu.CompilerParams(needs_layout_passes=False)


def _wid():
    return lax.axis_index("s") * 2 + lax.axis_index("c")


def _zero_ref(ref, n):
    zeros = jnp.zeros((16,), jnp.int32)

    def body(i, _):
        ref[pl.ds(i * 16, 16)] = zeros
        return 0

    lax.fori_loop(0, n // 16, body, 0, unroll=8)


def _hist_scan(z_hbm, base, buf0, buf1, sem0, sem1, process_chunk):
    def start(c, buf, sem):
        pltpu.async_copy(z_hbm.at[pl.ds(base + c * _CHUNK, _CHUNK)], buf, sem)

    def wait(buf, sem):
        pltpu.make_async_copy(z_hbm.at[pl.ds(base, _CHUNK)], buf, sem).wait()

    start(0, buf0, sem0)

    def pair_body(i, _):
        start(2 * i + 1, buf1, sem1)
        wait(buf0, sem0)
        process_chunk(buf0)

        @pl.when(i + 1 < _NPAIR)
        def _():
            start(2 * i + 2, buf0, sem0)

        wait(buf1, sem1)
        process_chunk(buf1)
        return 0

    lax.fori_loop(0, _NPAIR, pair_body, 0)


@functools.partial(
    pl.kernel,
    out_type=jax.ShapeDtypeStruct((_NW, _H1), jnp.int32),
    mesh=_mesh,
    compiler_params=_sc_params,
    scratch_types=[
        pltpu.VMEM((_CHUNK,), jnp.int32),
        pltpu.VMEM((_CHUNK,), jnp.int32),
        pltpu.VMEM((_H1,), jnp.int32),
        pltpu.SemaphoreType.DMA,
        pltpu.SemaphoreType.DMA,
    ],
)
def _sc_hist1(z_hbm, out_hbm, buf0, buf1, hist, sem0, sem1):
    wid = _wid()
    base = wid * _PER_W
    _zero_ref(hist, _H1)
    ones = jnp.ones((16,), jnp.int32)
    mask31 = jnp.int32(0x7FFFFFFF)

    def process_chunk(buf):
        def vec_body(i, _):
            b = buf[pl.ds(i * 16, 16)] & mask31
            idx = lax.shift_right_logical(b, 15)
            plsc.addupdate_scatter(hist, [idx], ones)
            return 0

        lax.fori_loop(0, _CHUNK // 16, vec_body, 0, unroll=8)

    _hist_scan(z_hbm, base, buf0, buf1, sem0, sem1, process_chunk)
    pltpu.sync_copy(hist, out_hbm.at[wid])


@functools.partial(
    pl.kernel,
    out_type=jax.ShapeDtypeStruct((_NW, _H2), jnp.int32),
    mesh=_mesh,
    compiler_params=_sc_params,
    scratch_types=[
        pltpu.VMEM((_CHUNK,), jnp.int32),
        pltpu.VMEM((_CHUNK,), jnp.int32),
        pltpu.VMEM((_H2,), jnp.int32),
        pltpu.VMEM((16,), jnp.int32),
        pltpu.SemaphoreType.DMA,
        pltpu.SemaphoreType.DMA,
    ],
)
def _sc_hist2(z_hbm, b1_hbm, out_hbm, buf0, buf1, hist, b1buf, sem0, sem1):
    wid = _wid()
    base = wid * _PER_W
    _zero_ref(hist, _H2)
    pltpu.sync_copy(b1_hbm, b1buf)
    b1 = b1buf[...]
    ones = jnp.ones((16,), jnp.int32)
    mask31 = jnp.int32(0x7FFFFFFF)
    mask15 = jnp.int32(0x7FFF)

    def process_chunk(buf):
        def vec_body(i, _):
            b = buf[pl.ds(i * 16, 16)] & mask31
            m = lax.shift_right_logical(b, 15) == b1
            idx = b & mask15
            plsc.addupdate_scatter(hist, [idx], ones, mask=m)
            return 0

        lax.fori_loop(0, _CHUNK // 16, vec_body, 0, unroll=8)

    _hist_scan(z_hbm, base, buf0, buf1, sem0, sem1, process_chunk)
    pltpu.sync_copy(hist, out_hbm.at[wid])


def _bucket_search(h_ref, need, nbuckets, nsteps):
    hs = jnp.sum(h_ref[...], axis=0)
    rows = nbuckets // 128
    idx = (lax.broadcasted_iota(jnp.int32, (rows, 128), 0) * 128
           + lax.broadcasted_iota(jnp.int32, (rows, 128), 1))

    def cnt(m):
        return jnp.sum(jnp.where(idx >= m, hs, 0))

    def body(t, lohi):
        lo, hi = lohi
        mid = (lo + hi) // 2
        p = cnt(mid) >= need
        return (jnp.where(p, mid, lo), jnp.where(p, hi, mid))

    lo, _ = lax.fori_loop(0, nsteps, body, (jnp.int32(0), jnp.int32(nbuckets)))
    return lo, cnt(lo + 1)


def _pick1_body(h_ref, b1_ref, r_ref):
    b1, above = _bucket_search(h_ref, jnp.int32(_KT), _H1, 16)
    b1_ref[...] = jnp.full((8, 128), b1, jnp.int32)
    r_ref[...] = jnp.full((8, 128), _KT - above, jnp.int32)


def _pick1(hist1):
    return pl.pallas_call(
        _pick1_body,
        in_specs=[pl.BlockSpec((_NW, _H1 // 128, 128), lambda: (0, 0, 0))],
        out_specs=[pl.BlockSpec((8, 128), lambda: (0, 0))] * 2,
        out_shape=[jax.ShapeDtypeStruct((8, 128), jnp.int32)] * 2,
    )(hist1.reshape(_NW, _H1 // 128, 128))


def _pick2_body(h_ref, b1_ref, r_ref, t_ref):
    lo, _ = _bucket_search(h_ref, r_ref[0, 0], _H2, 15)
    t = lax.shift_left(b1_ref[0, 0], 15) | lo
    t_ref[...] = jnp.full((8, 128), t, jnp.int32)


def _pick2(hist2, b1s, rs):
    return pl.pallas_call(
        _pick2_body,
        in_specs=[
            pl.BlockSpec((_NW, _H2 // 128, 128), lambda: (0, 0, 0)),
            pl.BlockSpec(memory_space=pltpu.MemorySpace.SMEM),
            pl.BlockSpec(memory_space=pltpu.MemorySpace.SMEM),
        ],
        out_specs=pl.BlockSpec((8, 128), lambda: (0, 0)),
        out_shape=jax.ShapeDtypeStruct((8, 128), jnp.int32),
    )(hist2.reshape(_NW, _H2 // 128, 128), b1s, rs)


def _enc_body(x_ref, w_ref, b_ref, z_ref):
    acc = lax.dot_general(
        x_ref[...], w_ref[...], (((1,), (1,)), ((), ())),
        preferred_element_type=jnp.float32)
    z_ref[...] = lax.bitcast_convert_type(acc + b_ref[...], jnp.int32)


def _encode(x, W_enc, b_enc):
    return pl.pallas_call(
        _enc_body,
        grid=(_D // _BN_ENC,),
        in_specs=[
            pl.BlockSpec((_B, _A), lambda j: (0, 0)),
            pl.BlockSpec((_BN_ENC, _A), lambda j: (j, 0)),
            pl.BlockSpec((1, _BN_ENC), lambda j: (0, j)),
        ],
        out_specs=pl.BlockSpec((_B, _BN_ENC), lambda j: (0, j)),
        out_shape=jax.ShapeDtypeStruct((_B, _D), jnp.int32),
    )(x, W_enc, b_enc.reshape(1, _D))


def _dec_body(t_ref, zb_ref, w_ref, b_ref, xhat_ref, sz_ref, mask_ref):
    j = pl.program_id(0)
    zb = zb_ref[...]
    z = lax.bitcast_convert_type(zb, jnp.float32)
    bits = zb & jnp.int32(0x7FFFFFFF)
    m = bits >= t_ref[0]
    sz = jnp.where(m, z, 0.0)
    sz_ref[...] = sz
    mask_ref[...] = m
    acc = lax.dot_general(
        sz, w_ref[...], (((1,), (1,)), ((), ())),
        preferred_element_type=jnp.float32)

    @pl.when(j == 0)
    def _():
        xhat_ref[...] = acc + b_ref[...]

    @pl.when(j > 0)
    def _():
        xhat_ref[...] += acc


def _decode(z_bits, W_dec, b_dec, thresh_bits):
    return pl.pallas_call(
        _dec_body,
        grid=(_D // _BD_DEC,),
        in_specs=[
            pl.BlockSpec(memory_space=pltpu.MemorySpace.SMEM),
            pl.BlockSpec((_B, _BD_DEC), lambda j: (0, j)),
            pl.BlockSpec((_A, _BD_DEC), lambda j: (0, j)),
            pl.BlockSpec((1, _A), lambda j: (0, 0)),
        ],
        out_specs=[
            pl.BlockSpec((_B, _A), lambda j: (0, 0)),
            pl.BlockSpec((_B, _BD_DEC), lambda j: (0, j)),
            pl.BlockSpec((_B, _BD_DEC), lambda j: (0, j)),
        ],
        out_shape=[
            jax.ShapeDtypeStruct((_B, _A), jnp.float32),
            jax.ShapeDtypeStruct((_B, _D), jnp.float32),
            jax.ShapeDtypeStruct((_B, _D), jnp.bool_),
        ],
    )(thresh_bits, z_bits, W_dec, b_dec.reshape(1, _A))


def kernel(x, W_enc, b_enc, W_dec, b_dec):
    z_bits = _encode(x, W_enc, b_enc)
    zf = z_bits.reshape(-1)
    hist1 = _sc_hist1(zf)
    b1_full, r_full = _pick1(hist1)
    b1vec = b1_full[0, :16]
    hist2 = _sc_hist2(zf, b1vec)
    t_full = _pick2(hist2, b1_full[:1, :1], r_full[:1, :1])
    x_hat, sparse_z, mask = _decode(z_bits, W_dec, b_dec, t_full[0, :1])
    return x_hat, sparse_z, mask


# speedup vs baseline: 72.0213x; 2.3212x over previous
import functools

import jax
import jax.numpy as jnp
from jax import lax
from jax.experimental import pallas as pl
from jax.experimental.pallas import tpu as pltpu
from jax.experimental.pallas import tpu_sc as plsc

_B = 2048
_A = 2048
_D = 16384
_K = 64
_KT = _K * _B
_N = _B * _D

_BN_ENC = 512
_BD_DEC = 256

_NW = 32
_PER_W = _N // _NW
_CHUNK = _D
_ROWS_W = _B // _NW
_NPAIR = _ROWS_W // 2
_H1 = 65536
_H2 = 32768

_mesh = plsc.VectorSubcoreMesh(core_axis_name="c", subcore_axis_name="s")
_sc_params = pltpu.CompilerParams(needs_layout_passes=False)


def _wid():
    return lax.axis_index("s") * 2 + lax.axis_index("c")


def _zero_ref(ref, n):
    zeros = jnp.zeros((16,), jnp.int32)

    def body(i, _):
        ref[pl.ds(i * 16, 16)] = zeros
        return 0

    lax.fori_loop(0, n // 16, body, 0, unroll=8)


def _hist_scan(z_hbm, base, buf0, buf1, sem0, sem1, process_chunk):
    def start(c, buf, sem):
        pltpu.async_copy(z_hbm.at[base + c], buf, sem)

    def wait(buf, sem):
        pltpu.make_async_copy(z_hbm.at[base], buf, sem).wait()

    start(0, buf0, sem0)

    def pair_body(i, _):
        start(2 * i + 1, buf1, sem1)
        wait(buf0, sem0)
        process_chunk(buf0)

        @pl.when(i + 1 < _NPAIR)
        def _():
            start(2 * i + 2, buf0, sem0)

        wait(buf1, sem1)
        process_chunk(buf1)
        return 0

    lax.fori_loop(0, _NPAIR, pair_body, 0)


@functools.partial(
    pl.kernel,
    out_type=jax.ShapeDtypeStruct((_NW, _H1), jnp.int32),
    mesh=_mesh,
    compiler_params=_sc_params,
    scratch_types=[
        pltpu.VMEM((_CHUNK,), jnp.int32),
        pltpu.VMEM((_CHUNK,), jnp.int32),
        pltpu.VMEM((_H1,), jnp.int32),
        pltpu.SemaphoreType.DMA,
        pltpu.SemaphoreType.DMA,
    ],
)
def _sc_hist1(z_hbm, out_hbm, buf0, buf1, hist, sem0, sem1):
    wid = _wid()
    base = wid * _ROWS_W
    _zero_ref(hist, _H1)
    ones = jnp.ones((16,), jnp.int32)
    mask31 = jnp.int32(0x7FFFFFFF)

    def process_chunk(buf):
        @plsc.parallel_loop(0, _CHUNK // 16, unroll=8)
        def vec_body(i):
            b = buf[pl.ds(i * 16, 16)] & mask31
            idx = lax.shift_right_logical(b, 15)
            plsc.addupdate_scatter(hist, [idx], ones)

    _hist_scan(z_hbm, base, buf0, buf1, sem0, sem1, process_chunk)
    pltpu.sync_copy(hist, out_hbm.at[wid])


@functools.partial(
    pl.kernel,
    out_type=jax.ShapeDtypeStruct((_NW, _H2), jnp.int32),
    mesh=_mesh,
    compiler_params=_sc_params,
    scratch_types=[
        pltpu.VMEM((_CHUNK,), jnp.int32),
        pltpu.VMEM((_CHUNK,), jnp.int32),
        pltpu.VMEM((_H2,), jnp.int32),
        pltpu.VMEM((16,), jnp.int32),
        pltpu.SemaphoreType.DMA,
        pltpu.SemaphoreType.DMA,
    ],
)
def _sc_hist2(z_hbm, b1_hbm, out_hbm, buf0, buf1, hist, b1buf, sem0, sem1):
    wid = _wid()
    base = wid * _ROWS_W
    _zero_ref(hist, _H2)
    pltpu.sync_copy(b1_hbm, b1buf)
    b1 = b1buf[...]
    ones = jnp.ones((16,), jnp.int32)
    mask31 = jnp.int32(0x7FFFFFFF)
    mask15 = jnp.int32(0x7FFF)

    def process_chunk(buf):
        @plsc.parallel_loop(0, _CHUNK // 16, unroll=8)
        def vec_body(i):
            b = buf[pl.ds(i * 16, 16)] & mask31
            m = lax.shift_right_logical(b, 15) == b1
            idx = b & mask15
            plsc.addupdate_scatter(hist, [idx], ones, mask=m)

    _hist_scan(z_hbm, base, buf0, buf1, sem0, sem1, process_chunk)
    pltpu.sync_copy(hist, out_hbm.at[wid])


def _bucket_search(h_ref, need, nbuckets, nsteps):
    hs = jnp.sum(h_ref[...], axis=0)
    rows = nbuckets // 128
    idx = (lax.broadcasted_iota(jnp.int32, (rows, 128), 0) * 128
           + lax.broadcasted_iota(jnp.int32, (rows, 128), 1))

    def cnt(m):
        return jnp.sum(jnp.where(idx >= m, hs, 0))

    def body(t, lohi):
        lo, hi = lohi
        mid = (lo + hi) // 2
        p = cnt(mid) >= need
        return (jnp.where(p, mid, lo), jnp.where(p, hi, mid))

    lo, _ = lax.fori_loop(0, nsteps, body, (jnp.int32(0), jnp.int32(nbuckets)))
    return lo, cnt(lo + 1)


def _pick1_body(h_ref, b1_ref, r_ref):
    b1, above = _bucket_search(h_ref, jnp.int32(_KT), _H1, 16)
    b1_ref[...] = jnp.full((8, 128), b1, jnp.int32)
    r_ref[...] = jnp.full((8, 128), _KT - above, jnp.int32)


def _pick1(hist1):
    return pl.pallas_call(
        _pick1_body,
        in_specs=[pl.BlockSpec((_NW, _H1 // 128, 128), lambda: (0, 0, 0))],
        out_specs=[pl.BlockSpec((8, 128), lambda: (0, 0))] * 2,
        out_shape=[jax.ShapeDtypeStruct((8, 128), jnp.int32)] * 2,
    )(hist1.reshape(_NW, _H1 // 128, 128))


def _pick2_body(h_ref, b1_ref, r_ref, t_ref):
    lo, _ = _bucket_search(h_ref, r_ref[0, 0], _H2, 15)
    t = lax.shift_left(b1_ref[0, 0], 15) | lo
    t_ref[...] = jnp.full((8, 128), t, jnp.int32)


def _pick2(hist2, b1s, rs):
    return pl.pallas_call(
        _pick2_body,
        in_specs=[
            pl.BlockSpec((_NW, _H2 // 128, 128), lambda: (0, 0, 0)),
            pl.BlockSpec(memory_space=pltpu.MemorySpace.SMEM),
            pl.BlockSpec(memory_space=pltpu.MemorySpace.SMEM),
        ],
        out_specs=pl.BlockSpec((8, 128), lambda: (0, 0)),
        out_shape=jax.ShapeDtypeStruct((8, 128), jnp.int32),
    )(hist2.reshape(_NW, _H2 // 128, 128), b1s, rs)


def _enc_body(x_ref, w_ref, b_ref, z_ref):
    acc = lax.dot_general(
        x_ref[...], w_ref[...], (((1,), (1,)), ((), ())),
        preferred_element_type=jnp.float32)
    z_ref[...] = lax.bitcast_convert_type(acc + b_ref[...], jnp.int32)


def _encode(x, W_enc, b_enc):
    return pl.pallas_call(
        _enc_body,
        grid=(_D // _BN_ENC,),
        in_specs=[
            pl.BlockSpec((_B, _A), lambda j: (0, 0)),
            pl.BlockSpec((_BN_ENC, _A), lambda j: (j, 0)),
            pl.BlockSpec((1, _BN_ENC), lambda j: (0, j)),
        ],
        out_specs=pl.BlockSpec((_B, _BN_ENC), lambda j: (0, j)),
        out_shape=jax.ShapeDtypeStruct((_B, _D), jnp.int32),
    )(x, W_enc, b_enc.reshape(1, _D))


def _dec_body(t_ref, zb_ref, w_ref, b_ref, xhat_ref, sz_ref, mask_ref):
    j = pl.program_id(0)
    zb = zb_ref[...]
    z = lax.bitcast_convert_type(zb, jnp.float32)
    bits = zb & jnp.int32(0x7FFFFFFF)
    m = bits >= t_ref[0]
    sz = jnp.where(m, z, 0.0)
    sz_ref[...] = sz
    mask_ref[...] = m
    acc = lax.dot_general(
        sz, w_ref[...], (((1,), (1,)), ((), ())),
        preferred_element_type=jnp.float32)

    @pl.when(j == 0)
    def _():
        xhat_ref[...] = acc + b_ref[...]

    @pl.when(j > 0)
    def _():
        xhat_ref[...] += acc


def _decode(z_bits, W_dec, b_dec, thresh_bits):
    return pl.pallas_call(
        _dec_body,
        grid=(_D // _BD_DEC,),
        in_specs=[
            pl.BlockSpec(memory_space=pltpu.MemorySpace.SMEM),
            pl.BlockSpec((_B, _BD_DEC), lambda j: (0, j)),
            pl.BlockSpec((_A, _BD_DEC), lambda j: (0, j)),
            pl.BlockSpec((1, _A), lambda j: (0, 0)),
        ],
        out_specs=[
            pl.BlockSpec((_B, _A), lambda j: (0, 0)),
            pl.BlockSpec((_B, _BD_DEC), lambda j: (0, j)),
            pl.BlockSpec((_B, _BD_DEC), lambda j: (0, j)),
        ],
        out_shape=[
            jax.ShapeDtypeStruct((_B, _A), jnp.float32),
            jax.ShapeDtypeStruct((_B, _D), jnp.float32),
            jax.ShapeDtypeStruct((_B, _D), jnp.bool_),
        ],
    )(thresh_bits, z_bits, W_dec, b_dec.reshape(1, _A))


def kernel(x, W_enc, b_enc, W_dec, b_dec):
    z_bits = _encode(x, W_enc, b_enc)
    hist1 = _sc_hist1(z_bits)
    b1_full, r_full = _pick1(hist1)
    b1vec = b1_full[0, :16]
    hist2 = _sc_hist2(z_bits, b1vec)
    t_full = _pick2(hist2, b1_full[:1, :1], r_full[:1, :1])
    x_hat, sparse_z, mask = _decode(z_bits, W_dec, b_dec, t_full[0, :1])
    return x_hat, sparse_z, mask


# speedup vs baseline: 74.2915x; 1.0315x over previous
import functools

import jax
import jax.numpy as jnp
from jax import lax
from jax.experimental import pallas as pl
from jax.experimental.pallas import tpu as pltpu
from jax.experimental.pallas import tpu_sc as plsc

_B = 2048
_A = 2048
_D = 16384
_K = 64
_KT = _K * _B
_N = _B * _D

_BN_ENC = 512
_BD_DEC = 512
_BM_DEC = 1024

_NW = 32
_PER_W = _N // _NW
_CHUNK = _D
_ROWS_W = _B // _NW
_NPAIR = _ROWS_W // 2
_H1 = 65536
_H2 = 32768

_mesh = plsc.VectorSubcoreMesh(core_axis_name="c", subcore_axis_name="s")
_sc_params = pltpu.CompilerParams(needs_layout_passes=False)


def _wid():
    return lax.axis_index("s") * 2 + lax.axis_index("c")


def _zero_ref(ref, n):
    zeros = jnp.zeros((16,), jnp.int32)

    def body(i, _):
        ref[pl.ds(i * 16, 16)] = zeros
        return 0

    lax.fori_loop(0, n // 16, body, 0, unroll=8)


def _hist_scan(z_hbm, base, buf0, buf1, sem0, sem1, process_chunk):
    def start(c, buf, sem):
        pltpu.async_copy(z_hbm.at[base + c], buf, sem)

    def wait(buf, sem):
        pltpu.make_async_copy(z_hbm.at[base], buf, sem).wait()

    start(0, buf0, sem0)

    def pair_body(i, _):
        start(2 * i + 1, buf1, sem1)
        wait(buf0, sem0)
        process_chunk(buf0)

        @pl.when(i + 1 < _NPAIR)
        def _():
            start(2 * i + 2, buf0, sem0)

        wait(buf1, sem1)
        process_chunk(buf1)
        return 0

    lax.fori_loop(0, _NPAIR, pair_body, 0)


@functools.partial(
    pl.kernel,
    out_type=jax.ShapeDtypeStruct((_NW, _H1), jnp.int32),
    mesh=_mesh,
    compiler_params=_sc_params,
    scratch_types=[
        pltpu.VMEM((_CHUNK,), jnp.int32),
        pltpu.VMEM((_CHUNK,), jnp.int32),
        pltpu.VMEM((_H1,), jnp.int32),
        pltpu.SemaphoreType.DMA,
        pltpu.SemaphoreType.DMA,
    ],
)
def _sc_hist1(z_hbm, out_hbm, buf0, buf1, hist, sem0, sem1):
    wid = _wid()
    base = wid * _ROWS_W
    _zero_ref(hist, _H1)
    ones = jnp.ones((16,), jnp.int32)
    mask31 = jnp.int32(0x7FFFFFFF)

    def process_chunk(buf):
        @plsc.parallel_loop(0, _CHUNK // 16, unroll=8)
        def vec_body(i):
            b = buf[pl.ds(i * 16, 16)] & mask31
            idx = lax.shift_right_logical(b, 15)
            plsc.addupdate_scatter(hist, [idx], ones)

    _hist_scan(z_hbm, base, buf0, buf1, sem0, sem1, process_chunk)
    pltpu.sync_copy(hist, out_hbm.at[wid])


@functools.partial(
    pl.kernel,
    out_type=jax.ShapeDtypeStruct((_NW, _H2), jnp.int32),
    mesh=_mesh,
    compiler_params=_sc_params,
    scratch_types=[
        pltpu.VMEM((_CHUNK,), jnp.int32),
        pltpu.VMEM((_CHUNK,), jnp.int32),
        pltpu.VMEM((_H2,), jnp.int32),
        pltpu.VMEM((16,), jnp.int32),
        pltpu.SemaphoreType.DMA,
        pltpu.SemaphoreType.DMA,
    ],
)
def _sc_hist2(z_hbm, b1_hbm, out_hbm, buf0, buf1, hist, b1buf, sem0, sem1):
    wid = _wid()
    base = wid * _ROWS_W
    _zero_ref(hist, _H2)
    pltpu.sync_copy(b1_hbm, b1buf)
    b1 = b1buf[...]
    ones = jnp.ones((16,), jnp.int32)
    mask31 = jnp.int32(0x7FFFFFFF)
    mask15 = jnp.int32(0x7FFF)

    def process_chunk(buf):
        @plsc.parallel_loop(0, _CHUNK // 16, unroll=8)
        def vec_body(i):
            b = buf[pl.ds(i * 16, 16)] & mask31
            m = lax.shift_right_logical(b, 15) == b1
            idx = b & mask15
            plsc.addupdate_scatter(hist, [idx], ones, mask=m)

    _hist_scan(z_hbm, base, buf0, buf1, sem0, sem1, process_chunk)
    pltpu.sync_copy(hist, out_hbm.at[wid])


def _bucket_search(h_ref, need, nbuckets, nsteps):
    hs = jnp.sum(h_ref[...], axis=0)
    rows = nbuckets // 128
    idx = (lax.broadcasted_iota(jnp.int32, (rows, 128), 0) * 128
           + lax.broadcasted_iota(jnp.int32, (rows, 128), 1))

    def cnt(m):
        return jnp.sum(jnp.where(idx >= m, hs, 0))

    def body(t, lohi):
        lo, hi = lohi
        mid = (lo + hi) // 2
        p = cnt(mid) >= need
        return (jnp.where(p, mid, lo), jnp.where(p, hi, mid))

    lo, _ = lax.fori_loop(0, nsteps, body, (jnp.int32(0), jnp.int32(nbuckets)))
    return lo, cnt(lo + 1)


def _pick1_body(h_ref, b1_ref, r_ref):
    b1, above = _bucket_search(h_ref, jnp.int32(_KT), _H1, 16)
    b1_ref[...] = jnp.full((8, 128), b1, jnp.int32)
    r_ref[...] = jnp.full((8, 128), _KT - above, jnp.int32)


def _pick1(hist1):
    return pl.pallas_call(
        _pick1_body,
        in_specs=[pl.BlockSpec((_NW, _H1 // 128, 128), lambda: (0, 0, 0))],
        out_specs=[pl.BlockSpec((8, 128), lambda: (0, 0))] * 2,
        out_shape=[jax.ShapeDtypeStruct((8, 128), jnp.int32)] * 2,
    )(hist1.reshape(_NW, _H1 // 128, 128))


def _pick2_body(h_ref, b1_ref, r_ref, t_ref):
    lo, _ = _bucket_search(h_ref, r_ref[0, 0], _H2, 15)
    t = lax.shift_left(b1_ref[0, 0], 15) | lo
    t_ref[...] = jnp.full((8, 128), t, jnp.int32)


def _pick2(hist2, b1s, rs):
    return pl.pallas_call(
        _pick2_body,
        in_specs=[
            pl.BlockSpec((_NW, _H2 // 128, 128), lambda: (0, 0, 0)),
            pl.BlockSpec(memory_space=pltpu.MemorySpace.SMEM),
            pl.BlockSpec(memory_space=pltpu.MemorySpace.SMEM),
        ],
        out_specs=pl.BlockSpec((8, 128), lambda: (0, 0)),
        out_shape=jax.ShapeDtypeStruct((8, 128), jnp.int32),
    )(hist2.reshape(_NW, _H2 // 128, 128), b1s, rs)


def _enc_body(x_ref, w_ref, b_ref, z_ref):
    acc = lax.dot_general(
        x_ref[...], w_ref[...], (((1,), (1,)), ((), ())),
        preferred_element_type=jnp.float32)
    z_ref[...] = lax.bitcast_convert_type(acc + b_ref[...], jnp.int32)


def _encode(x, W_enc, b_enc):
    return pl.pallas_call(
        _enc_body,
        grid=(_D // _BN_ENC,),
        in_specs=[
            pl.BlockSpec((_B, _A), lambda j: (0, 0)),
            pl.BlockSpec((_BN_ENC, _A), lambda j: (j, 0)),
            pl.BlockSpec((1, _BN_ENC), lambda j: (0, j)),
        ],
        out_specs=pl.BlockSpec((_B, _BN_ENC), lambda j: (0, j)),
        out_shape=jax.ShapeDtypeStruct((_B, _D), jnp.int32),
    )(x, W_enc, b_enc.reshape(1, _D))


def _dec_body(t_ref, zb_ref, w_ref, b_ref, xhat_ref, sz_ref, mask_ref):
    j = pl.program_id(1)
    zb = zb_ref[...]
    z = lax.bitcast_convert_type(zb, jnp.float32)
    bits = zb & jnp.int32(0x7FFFFFFF)
    m = bits >= t_ref[0]
    sz = jnp.where(m, z, 0.0)
    sz_ref[...] = sz
    mask_ref[...] = m
    acc = lax.dot_general(
        sz, w_ref[...], (((1,), (1,)), ((), ())),
        preferred_element_type=jnp.float32)

    @pl.when(j == 0)
    def _():
        xhat_ref[...] = acc + b_ref[...]

    @pl.when(j > 0)
    def _():
        xhat_ref[...] += acc


def _decode(z_bits, W_dec, b_dec, thresh_bits):
    return pl.pallas_call(
        _dec_body,
        grid=(_B // _BM_DEC, _D // _BD_DEC),
        in_specs=[
            pl.BlockSpec(memory_space=pltpu.MemorySpace.SMEM),
            pl.BlockSpec((_BM_DEC, _BD_DEC), lambda i, j: (i, j)),
            pl.BlockSpec((_A, _BD_DEC), lambda i, j: (0, j)),
            pl.BlockSpec((1, _A), lambda i, j: (0, 0)),
        ],
        out_specs=[
            pl.BlockSpec((_BM_DEC, _A), lambda i, j: (i, 0)),
            pl.BlockSpec((_BM_DEC, _BD_DEC), lambda i, j: (i, j)),
            pl.BlockSpec((_BM_DEC, _BD_DEC), lambda i, j: (i, j)),
        ],
        out_shape=[
            jax.ShapeDtypeStruct((_B, _A), jnp.float32),
            jax.ShapeDtypeStruct((_B, _D), jnp.float32),
            jax.ShapeDtypeStruct((_B, _D), jnp.bool_),
        ],
    )(thresh_bits, z_bits, W_dec, b_dec.reshape(1, _A))


def kernel(x, W_enc, b_enc, W_dec, b_dec):
    z_bits = _encode(x, W_enc, b_enc)
    hist1 = _sc_hist1(z_bits)
    b1_full, r_full = _pick1(hist1)
    b1vec = b1_full[0, :16]
    hist2 = _sc_hist2(z_bits, b1vec)
    t_full = _pick2(hist2, b1_full[:1, :1], r_full[:1, :1])
    x_hat, sparse_z, mask = _decode(z_bits, W_dec, b_dec, t_full[0, :1])
    return x_hat, sparse_z, mask
